# Initial kernel scaffold; baseline (speedup 1.0000x reference)
import jax
import jax.numpy as jnp
from jax.experimental import pallas as pl


def kernel(original_features, sampled_features, conv_w, conv_b, rconv_w, rconv_b):
    raise NotImplementedError("write your pallas kernel here")


# speedup vs baseline: 3.5943x; 3.5943x over previous
import jax
import jax.numpy as jnp
from jax import lax
from jax.experimental import pallas as pl
from jax.experimental.pallas import tpu as pltpu
from jax.experimental.pallas import tpu_sc as plsc

_B, _C, _N, _K = 8, 16, 65536, 10
_NC, _NS = 2, 16
_NW = _NC * _NS
_P = 256
_NPW = _N // _NW
_NBLK = _NPW // _P
_TOT = _B * _NBLK
_L = 16


def _sigmoid(x):
    return 1.0 / (1.0 + jnp.exp(-x))


def _full(v):
    return jnp.full((_L,), v, jnp.int32)


def _bf16(x):
    b = jax.lax.bitcast_convert_type(x, jnp.int32)
    r = b + 0x7FFF + (jax.lax.shift_right_logical(b, 16) & 1)
    return jax.lax.bitcast_convert_type(
        r & jnp.int32(-65536), jnp.float32)


_GDN = jax.lax.GatherDimensionNumbers(
    offset_dims=(), collapsed_slice_dims=(0,), start_index_map=(0,))


def _bcast(vec, c):
    return jax.lax.gather(
        vec, _full(c)[:, None], dimension_numbers=_GDN, slice_sizes=(1,),
        mode=jax.lax.GatherScatterMode.PROMISE_IN_BOUNDS)


def _body(samp_hbm, orig_hbm, par_hbm, out_hbm,
          sbuf, obuf, outbuf, ubuf, wbuf, dbuf, rdbuf, pbuf, sem):
    wid = lax.axis_index("s") * _NC + lax.axis_index("c")

    pltpu.sync_copy(par_hbm, pbuf)


    def block(t, carry):
        b = t >> 3
        n0 = wid * _NPW + (t & 7) * _P
        pltpu.sync_copy(samp_hbm.at[b, :, pl.ds(n0 * _K, _P * _K)], sbuf)
        pltpu.sync_copy(orig_hbm.at[b, :, pl.ds(n0, _P)], obuf)

        def phase_a(j, c2):
            base = j * _L
            whi_v = _bf16(pbuf[16:32])
            rhi_v = pbuf[48:64]
            up = [jnp.zeros((_L,), jnp.float32) for _ in range(4)]
            wp = [jnp.zeros((_L,), jnp.float32) for _ in range(4)]
            for c in range(_C):
                v = sbuf[c, pl.ds(base, _L)]
                up[c % 4] = up[c % 4] + _bcast(whi_v, c) * _bf16(v)
                wp[c % 4] = wp[c % 4] + _bcast(rhi_v, c) * v
            ubuf[pl.ds(base, _L)] = (up[0] + up[1]) + (up[2] + up[3])
            wbuf[pl.ds(base, _L)] = (wp[0] + wp[1]) + (wp[2] + wp[3])
            return c2

        lax.fori_loop(0, _P * _K // _L, phase_a, 0)

        def phase_a2(i, c2):
            base = i * _L
            wlo_v = _bf16(pbuf[0:16])
            rlo_v = pbuf[32:48]
            dp = [jnp.zeros((_L,), jnp.float32) for _ in range(4)]
            rp = [jnp.zeros((_L,), jnp.float32) for _ in range(4)]
            for c in range(_C):
                ov = obuf[c, pl.ds(base, _L)]
                dp[c % 4] = dp[c % 4] + _bcast(wlo_v, c) * _bf16(ov)
                rp[c % 4] = rp[c % 4] + _bcast(rlo_v, c) * ov
            dbuf[pl.ds(base, _L)] = (dp[0] + dp[1]) + (dp[2] + dp[3])
            rdbuf[pl.ds(base, _L)] = (rp[0] + rp[1]) + (rp[2] + rp[3])
            return c2

        lax.fori_loop(0, _P // _L, phase_a2, 0)

        def phase_bc(i, c2):
            p16 = i * _L
            rb = plsc.load_gather(pbuf, [_full(65)])
            d = dbuf[pl.ds(p16, _L)]
            rd = rdbuf[pl.ds(p16, _L)]
            gidx = (lax.iota(jnp.int32, _L) + p16) * _K
            s = []
            g = []
            for k in range(_K):
                uk = plsc.load_gather(ubuf, [gidx + k])
                wk = plsc.load_gather(wbuf, [gidx + k])
                s.append(uk + d)
                g.append(_sigmoid(wk + rd + rb))
            rank = [jnp.zeros((_L,), jnp.float32) for _ in range(_K)]
            one = jnp.ones((_L,), jnp.float32)
            for k in range(_K):
                for j in range(k):
                    cf = jnp.where(s[j] >= s[k], 1.0, 0.0)
                    rank[k] = rank[k] + cf
                    rank[j] = rank[j] + (one - cf)
            for k in range(_K):
                g[k] = jnp.where(rank[k] < 4.0, g[k], 0.0)

            def chan(c, c3):
                cvec = _full(0) + c
                svs = [plsc.load_gather(sbuf, [cvec, gidx + k])
                       for k in range(_K)]
                t2 = [g[2 * m] * svs[2 * m] +
                      g[2 * m + 1] * svs[2 * m + 1] for m in range(5)]
                acc = ((t2[0] + t2[1]) + (t2[2] + t2[3]) +
                       (t2[4] + obuf[c, pl.ds(p16, _L)]))
                outbuf[c, pl.ds(p16, _L)] = acc
                return c3

            lax.fori_loop(0, _C, chan, 0)
            return c2

        lax.fori_loop(0, _P // _L, phase_bc, 0)

        pltpu.sync_copy(outbuf, out_hbm.at[b, :, pl.ds(n0, _P)])
        return carry

    lax.fori_loop(0, _TOT, block, 0)


@jax.jit
def _run(samp_flat, orig, params):
    mesh = plsc.VectorSubcoreMesh(
        core_axis_name="c", subcore_axis_name="s",
        num_cores=_NC, num_subcores=_NS)
    return pl.kernel(
        _body,
        out_type=jax.ShapeDtypeStruct((_B, _C, _N), jnp.float32),
        mesh=mesh,
        compiler_params=pltpu.CompilerParams(needs_layout_passes=False),
        scratch_types=[
            pltpu.VMEM((_C, _P * _K), jnp.float32),
            pltpu.VMEM((_C, _P), jnp.float32),
            pltpu.VMEM((_C, _P), jnp.float32),
            pltpu.VMEM((_P * _K,), jnp.float32),
            pltpu.VMEM((_P * _K,), jnp.float32),
            pltpu.VMEM((_P,), jnp.float32),
            pltpu.VMEM((_P,), jnp.float32),
            pltpu.VMEM((80,), jnp.float32),
            pltpu.SemaphoreType.DMA,
        ],
    )(samp_flat, orig, params)


def kernel(original_features, sampled_features, conv_w, conv_b,
           rconv_w, rconv_b):
    samp_flat = sampled_features.reshape(_B, _C, _N * _K)
    params = jnp.concatenate([
        conv_w[0], rconv_w[0], conv_b, rconv_b,
        jnp.zeros((14,), jnp.float32)])
    return _run(samp_flat, original_features, params)


# speedup vs baseline: 3.6662x; 1.0200x over previous
import jax
import jax.numpy as jnp
from jax import lax
from jax.experimental import pallas as pl
from jax.experimental.pallas import tpu as pltpu
from jax.experimental.pallas import tpu_sc as plsc

_B, _C, _N, _K = 8, 16, 65536, 10
_NC, _NS = 2, 16
_NW = _NC * _NS
_P = 256
_NPW = _N // _NW
_NBLK = _NPW // _P
_TOT = _B * _NBLK
_L = 16


def _sigmoid(x):
    return 1.0 / (1.0 + jnp.exp(-x))


def _full(v):
    return jnp.full((_L,), v, jnp.int32)


def _bf16(x):
    b = jax.lax.bitcast_convert_type(x, jnp.int32)
    r = b + 0x7FFF + (jax.lax.shift_right_logical(b, 16) & 1)
    return jax.lax.bitcast_convert_type(
        r & jnp.int32(-65536), jnp.float32)


_GDN = jax.lax.GatherDimensionNumbers(
    offset_dims=(), collapsed_slice_dims=(0,), start_index_map=(0,))


def _bcast(vec, c):
    return jax.lax.gather(
        vec, _full(c)[:, None], dimension_numbers=_GDN, slice_sizes=(1,),
        mode=jax.lax.GatherScatterMode.PROMISE_IN_BOUNDS)


def _body(samp_hbm, orig_hbm, par_hbm, out_hbm,
          sbuf, obuf, outbuf, ubuf, wbuf, dbuf, rdbuf, pbuf, dsem):
    wid = lax.axis_index("s") * _NC + lax.axis_index("c")

    pltpu.sync_copy(par_hbm, pbuf)


    def slab(t):
        b = t >> 3
        n0 = wid * _NPW + (t & 7) * _P
        return samp_hbm.at[b, :, pl.ds(n0 * _K, _P * _K)]

    def block(t, carry):
        b = t >> 3
        par = t & 1
        n0 = wid * _NPW + (t & 7) * _P
        pltpu.make_async_copy(slab(t), sbuf.at[par], dsem.at[par]).wait()

        @pl.when(t + 1 < _TOT)
        def _():
            pltpu.async_copy(slab(t + 1), sbuf.at[1 - par],
                             dsem.at[1 - par])

        pltpu.sync_copy(orig_hbm.at[b, :, pl.ds(n0, _P)], obuf)

        def phase_a(j, c2):
            base = j * _L
            whi_v = _bf16(pbuf[16:32])
            rhi_v = pbuf[48:64]
            up = [jnp.zeros((_L,), jnp.float32) for _ in range(4)]
            wp = [jnp.zeros((_L,), jnp.float32) for _ in range(4)]
            for c in range(_C):
                v = sbuf[par, c, pl.ds(base, _L)]
                up[c % 4] = up[c % 4] + _bcast(whi_v, c) * _bf16(v)
                wp[c % 4] = wp[c % 4] + _bcast(rhi_v, c) * v
            ubuf[pl.ds(base, _L)] = (up[0] + up[1]) + (up[2] + up[3])
            wbuf[pl.ds(base, _L)] = (wp[0] + wp[1]) + (wp[2] + wp[3])
            return c2

        lax.fori_loop(0, _P * _K // _L, phase_a, 0)

        def phase_a2(i, c2):
            base = i * _L
            wlo_v = _bf16(pbuf[0:16])
            rlo_v = pbuf[32:48]
            dp = [jnp.zeros((_L,), jnp.float32) for _ in range(4)]
            rp = [jnp.zeros((_L,), jnp.float32) for _ in range(4)]
            for c in range(_C):
                ov = obuf[c, pl.ds(base, _L)]
                dp[c % 4] = dp[c % 4] + _bcast(wlo_v, c) * _bf16(ov)
                rp[c % 4] = rp[c % 4] + _bcast(rlo_v, c) * ov
            dbuf[pl.ds(base, _L)] = (dp[0] + dp[1]) + (dp[2] + dp[3])
            rdbuf[pl.ds(base, _L)] = (rp[0] + rp[1]) + (rp[2] + rp[3])
            return c2

        lax.fori_loop(0, _P // _L, phase_a2, 0)

        def phase_bc(i, c2):
            p16 = i * _L
            rb = plsc.load_gather(pbuf, [_full(65)])
            d = dbuf[pl.ds(p16, _L)]
            rd = rdbuf[pl.ds(p16, _L)]
            gidx = (lax.iota(jnp.int32, _L) + p16) * _K
            s = []
            g = []
            for k in range(_K):
                uk = plsc.load_gather(ubuf, [gidx + k])
                wk = plsc.load_gather(wbuf, [gidx + k])
                s.append(uk + d)
                g.append(_sigmoid(wk + rd + rb))
            rank = [jnp.zeros((_L,), jnp.float32) for _ in range(_K)]
            one = jnp.ones((_L,), jnp.float32)
            for k in range(_K):
                for j in range(k):
                    cf = jnp.where(s[j] >= s[k], 1.0, 0.0)
                    rank[k] = rank[k] + cf
                    rank[j] = rank[j] + (one - cf)
            for k in range(_K):
                g[k] = jnp.where(rank[k] < 4.0, g[k], 0.0)

            def chan(c, c3):
                cvec = _full(0) + c
                pvec = _full(0) + par
                svs = [plsc.load_gather(sbuf, [pvec, cvec, gidx + k])
                       for k in range(_K)]
                t2 = [g[2 * m] * svs[2 * m] +
                      g[2 * m + 1] * svs[2 * m + 1] for m in range(5)]
                acc = ((t2[0] + t2[1]) + (t2[2] + t2[3]) +
                       (t2[4] + obuf[c, pl.ds(p16, _L)]))
                outbuf[c, pl.ds(p16, _L)] = acc
                return c3

            lax.fori_loop(0, _C, chan, 0)
            return c2

        lax.fori_loop(0, _P // _L, phase_bc, 0)

        pltpu.sync_copy(outbuf, out_hbm.at[b, :, pl.ds(n0, _P)])
        return carry

    pltpu.async_copy(slab(0), sbuf.at[0], dsem.at[0])
    lax.fori_loop(0, _TOT, block, 0)


@jax.jit
def _run(samp_flat, orig, params):
    mesh = plsc.VectorSubcoreMesh(
        core_axis_name="c", subcore_axis_name="s",
        num_cores=_NC, num_subcores=_NS)
    return pl.kernel(
        _body,
        out_type=jax.ShapeDtypeStruct((_B, _C, _N), jnp.float32),
        mesh=mesh,
        compiler_params=pltpu.CompilerParams(needs_layout_passes=False),
        scratch_types=[
            pltpu.VMEM((2, _C, _P * _K), jnp.float32),
            pltpu.VMEM((_C, _P), jnp.float32),
            pltpu.VMEM((_C, _P), jnp.float32),
            pltpu.VMEM((_P * _K,), jnp.float32),
            pltpu.VMEM((_P * _K,), jnp.float32),
            pltpu.VMEM((_P,), jnp.float32),
            pltpu.VMEM((_P,), jnp.float32),
            pltpu.VMEM((80,), jnp.float32),
            pltpu.SemaphoreType.DMA((2,)),
        ],
    )(samp_flat, orig, params)


def kernel(original_features, sampled_features, conv_w, conv_b,
           rconv_w, rconv_b):
    samp_flat = sampled_features.reshape(_B, _C, _N * _K)
    params = jnp.concatenate([
        conv_w[0], rconv_w[0], conv_b, rconv_b,
        jnp.zeros((14,), jnp.float32)])
    return _run(samp_flat, original_features, params)


# speedup vs baseline: 3.6787x; 1.0034x over previous
import jax
import jax.numpy as jnp
from jax import lax
from jax.experimental import pallas as pl
from jax.experimental.pallas import tpu as pltpu
from jax.experimental.pallas import tpu_sc as plsc

_B, _C, _N, _K = 8, 16, 65536, 10
_NC, _NS = 2, 16
_NW = _NC * _NS
_P = 256
_NPW = _N // _NW
_NBLK = _NPW // _P
_TOT = _B * _NBLK
_L = 16


def _sigmoid(x):
    return 1.0 / (1.0 + jnp.exp(-x))


def _full(v):
    return jnp.full((_L,), v, jnp.int32)


def _bf16(x):
    b = jax.lax.bitcast_convert_type(x, jnp.int32)
    r = b + 0x7FFF + (jax.lax.shift_right_logical(b, 16) & 1)
    return jax.lax.bitcast_convert_type(
        r & jnp.int32(-65536), jnp.float32)


_GDN = jax.lax.GatherDimensionNumbers(
    offset_dims=(), collapsed_slice_dims=(0,), start_index_map=(0,))


def _bcast(vec, c):
    return jax.lax.gather(
        vec, _full(c)[:, None], dimension_numbers=_GDN, slice_sizes=(1,),
        mode=jax.lax.GatherScatterMode.PROMISE_IN_BOUNDS)


def _body(samp_hbm, orig_hbm, par_hbm, out_hbm,
          sbuf, obuf, outbuf, ubuf, wbuf, dbuf, rdbuf, pbuf, dsem):
    wid = lax.axis_index("s") * _NC + lax.axis_index("c")

    pltpu.sync_copy(par_hbm, pbuf)


    def slab(t):
        b = t >> 3
        n0 = wid * _NPW + (t & 7) * _P
        return samp_hbm.at[b, :, pl.ds(n0 * _K, _P * _K)]

    def block(t, carry):
        b = t >> 3
        par = t & 1
        n0 = wid * _NPW + (t & 7) * _P
        pltpu.make_async_copy(slab(t), sbuf.at[par], dsem.at[par]).wait()

        @pl.when(t + 1 < _TOT)
        def _():
            pltpu.async_copy(slab(t + 1), sbuf.at[1 - par],
                             dsem.at[1 - par])

        pltpu.sync_copy(orig_hbm.at[b, :, pl.ds(n0, _P)], obuf)

        def phase_a(j, c2):
            base = j * _L
            whi_v = _bf16(pbuf[16:32])
            rhi_v = pbuf[48:64]
            up = [jnp.zeros((_L,), jnp.float32) for _ in range(4)]
            wp = [jnp.zeros((_L,), jnp.float32) for _ in range(4)]
            for c in range(_C):
                v = sbuf[par, c, pl.ds(base, _L)]
                up[c % 4] = up[c % 4] + _bcast(whi_v, c) * _bf16(v)
                wp[c % 4] = wp[c % 4] + _bcast(rhi_v, c) * v
            ubuf[pl.ds(base, _L)] = (up[0] + up[1]) + (up[2] + up[3])
            wbuf[pl.ds(base, _L)] = (wp[0] + wp[1]) + (wp[2] + wp[3])
            return c2

        lax.fori_loop(0, _P * _K // _L, phase_a, 0)

        def phase_a2(i, c2):
            base = i * _L
            wlo_v = _bf16(pbuf[0:16])
            rlo_v = pbuf[32:48]
            dp = [jnp.zeros((_L,), jnp.float32) for _ in range(4)]
            rp = [jnp.zeros((_L,), jnp.float32) for _ in range(4)]
            for c in range(_C):
                ov = obuf[c, pl.ds(base, _L)]
                dp[c % 4] = dp[c % 4] + _bcast(wlo_v, c) * _bf16(ov)
                rp[c % 4] = rp[c % 4] + _bcast(rlo_v, c) * ov
            dbuf[pl.ds(base, _L)] = (dp[0] + dp[1]) + (dp[2] + dp[3])
            rdbuf[pl.ds(base, _L)] = (rp[0] + rp[1]) + (rp[2] + rp[3])
            return c2

        lax.fori_loop(0, _P // _L, phase_a2, 0)

        def phase_bc(i, c2):
            p16 = i * _L
            rb = plsc.load_gather(pbuf, [_full(65)])
            d = dbuf[pl.ds(p16, _L)]
            rd = rdbuf[pl.ds(p16, _L)]
            gidx = (lax.iota(jnp.int32, _L) + p16) * _K
            s = []
            g = []
            for k in range(_K):
                uk = plsc.load_gather(ubuf, [gidx + k])
                wk = plsc.load_gather(wbuf, [gidx + k])
                s.append(uk + d)
                g.append(_sigmoid(wk + rd + rb))
            rank = [jnp.zeros((_L,), jnp.float32) for _ in range(_K)]
            one = jnp.ones((_L,), jnp.float32)
            for k in range(_K):
                for j in range(k):
                    cf = jnp.where(s[j] >= s[k], 1.0, 0.0)
                    rank[k] = rank[k] + cf
                    rank[j] = rank[j] + (one - cf)
            sel_idx = []
            sel_g = []
            for r in range(4):
                ivec = jnp.zeros((_L,), jnp.float32)
                gvec = jnp.zeros((_L,), jnp.float32)
                for k in range(_K):
                    m = rank[k] == float(r)
                    ivec = ivec + jnp.where(m, float(k), 0.0)
                    gvec = gvec + jnp.where(m, g[k], 0.0)
                sel_idx.append(gidx + ivec.astype(jnp.int32))
                sel_g.append(gvec)

            def chan(c, c3):
                cvec = _full(0) + c
                pvec = _full(0) + par
                svs = [plsc.load_gather(sbuf, [pvec, cvec, sel_idx[r]])
                       for r in range(4)]
                acc = ((sel_g[0] * svs[0] + sel_g[1] * svs[1]) +
                       (sel_g[2] * svs[2] + sel_g[3] * svs[3]) +
                       obuf[c, pl.ds(p16, _L)])
                outbuf[c, pl.ds(p16, _L)] = acc
                return c3

            lax.fori_loop(0, _C, chan, 0)
            return c2

        lax.fori_loop(0, _P // _L, phase_bc, 0)

        pltpu.sync_copy(outbuf, out_hbm.at[b, :, pl.ds(n0, _P)])
        return carry

    pltpu.async_copy(slab(0), sbuf.at[0], dsem.at[0])
    lax.fori_loop(0, _TOT, block, 0)


@jax.jit
def _run(samp_flat, orig, params):
    mesh = plsc.VectorSubcoreMesh(
        core_axis_name="c", subcore_axis_name="s",
        num_cores=_NC, num_subcores=_NS)
    return pl.kernel(
        _body,
        out_type=jax.ShapeDtypeStruct((_B, _C, _N), jnp.float32),
        mesh=mesh,
        compiler_params=pltpu.CompilerParams(needs_layout_passes=False),
        scratch_types=[
            pltpu.VMEM((2, _C, _P * _K), jnp.float32),
            pltpu.VMEM((_C, _P), jnp.float32),
            pltpu.VMEM((_C, _P), jnp.float32),
            pltpu.VMEM((_P * _K,), jnp.float32),
            pltpu.VMEM((_P * _K,), jnp.float32),
            pltpu.VMEM((_P,), jnp.float32),
            pltpu.VMEM((_P,), jnp.float32),
            pltpu.VMEM((80,), jnp.float32),
            pltpu.SemaphoreType.DMA((2,)),
        ],
    )(samp_flat, orig, params)


def kernel(original_features, sampled_features, conv_w, conv_b,
           rconv_w, rconv_b):
    samp_flat = sampled_features.reshape(_B, _C, _N * _K)
    params = jnp.concatenate([
        conv_w[0], rconv_w[0], conv_b, rconv_b,
        jnp.zeros((14,), jnp.float32)])
    return _run(samp_flat, original_features, params)
